# baseline (speedup 1.0000x reference)
import jax
import jax.numpy as jnp
from jax import lax
from jax.experimental import pallas as pl
from jax.experimental.pallas import tpu as pltpu

N = 32
IN_CH = 512
HIDDEN = 256
HEADS = 4
OUT_CH = HIDDEN // HEADS
SPATIAL = 56 * 56
NUM_AU = 32
ROWS_PER_STEP = 512


def _fused_body(x_ref, eiT_ref, wlin_ref, att_ref, bias_ref, wcls_ref,
                bcls_ref, out_ref, acc_ref):
    g = pl.program_id(0)
    num_steps = pl.num_programs(0)
    imgs_per_step = ROWS_PER_STEP // IN_CH

    s = jnp.sum(x_ref[...], axis=1)
    acc_ref[pl.ds(g * imgs_per_step, imgs_per_step), :] = s.reshape(
        imgs_per_step, IN_CH)

    @pl.when(g == num_steps - 1)
    def _epilogue():
        x = acc_ref[...] * (1.0 / SPATIAL)
        h = jnp.dot(x, wlin_ref[...],
                    preferred_element_type=jnp.float32)
        hh = h.reshape(N, HEADS, OUT_CH)
        att_src = att_ref[0:HEADS, :]
        att_dst = att_ref[HEADS:2 * HEADS, :]
        a_src = jnp.sum(hh * att_src[None, :, :], axis=-1)
        a_dst = jnp.sum(hh * att_dst[None, :, :], axis=-1)

        e = a_src[:, None, :] + a_dst[None, :, :]
        e = jnp.where(e >= 0, e, 0.2 * e)

        src = eiT_ref[:, 0:1]
        dst = eiT_ref[:, 1:2]
        num_edges = eiT_ref.shape[0]
        ids = lax.broadcasted_iota(jnp.int32, (num_edges, N), 1)
        oh_s = (src == ids).astype(jnp.float32)
        oh_d = (dst == ids).astype(jnp.float32)
        cmat = lax.dot_general(oh_s, oh_d, (((0,), (0,)), ((), ())),
                               preferred_element_type=jnp.float32)
        present = (cmat > 0.0).astype(jnp.float32)

        e_m = e + (present[:, :, None] - 1.0) * jnp.float32(1e30)
        mx = jnp.max(e_m, axis=0)
        ee = jnp.exp(jnp.minimum(e - mx[None, :, :], 0.0))
        ee = ee * cmat[:, :, None]
        denom = jnp.sum(ee, axis=0) + jnp.float32(1e-16)

        outs = []
        for hd in range(HEADS):
            w = ee[:, :, hd]
            num = lax.dot_general(w, h[:, hd * OUT_CH:(hd + 1) * OUT_CH],
                                  (((0,), (0,)), ((), ())),
                                  preferred_element_type=jnp.float32)
            outs.append(num / denom[:, hd][:, None])
        gat = jnp.concatenate(outs, axis=1) + bias_ref[...]
        act = gat * jax.nn.sigmoid(gat)
        logit = jnp.dot(act, wcls_ref[...],
                        preferred_element_type=jnp.float32) + bcls_ref[...]
        out_ref[...] = logit


def kernel(roi_feats, edge_index, W_lin, att_src, att_dst, bias_gat,
           W_cls, b_cls):
    x2d = roi_feats.reshape(N * IN_CH, SPATIAL)
    eiT = edge_index.T
    att = jnp.concatenate([att_src, att_dst], axis=0)
    bias2d = bias_gat.reshape(1, HIDDEN)
    bcls2d = b_cls.reshape(1, NUM_AU)
    grid = (x2d.shape[0] // ROWS_PER_STEP,)

    out = pl.pallas_call(
        _fused_body,
        grid=grid,
        in_specs=[
            pl.BlockSpec((ROWS_PER_STEP, SPATIAL), lambda g: (g, 0)),
            pl.BlockSpec(eiT.shape, lambda g: (0, 0)),
            pl.BlockSpec(W_lin.shape, lambda g: (0, 0)),
            pl.BlockSpec(att.shape, lambda g: (0, 0)),
            pl.BlockSpec(bias2d.shape, lambda g: (0, 0)),
            pl.BlockSpec(W_cls.shape, lambda g: (0, 0)),
            pl.BlockSpec(bcls2d.shape, lambda g: (0, 0)),
        ],
        out_specs=pl.BlockSpec((N, NUM_AU), lambda g: (0, 0)),
        out_shape=jax.ShapeDtypeStruct((N, NUM_AU), jnp.float32),
        scratch_shapes=[pltpu.VMEM((N, IN_CH), jnp.float32)],
        compiler_params=pltpu.CompilerParams(
            dimension_semantics=("arbitrary",)),
    )(x2d, eiT, W_lin, att, bias2d, W_cls, bcls2d)
    return out


# speedup vs baseline: 1.8294x; 1.8294x over previous
import jax
import jax.numpy as jnp
from jax import lax
from jax.experimental import pallas as pl
from jax.experimental.pallas import tpu as pltpu

N = 32
IN_CH = 512
HIDDEN = 256
HEADS = 4
OUT_CH = HIDDEN // HEADS
SPATIAL = 56 * 56
NUM_AU = 32
ROWS_PER_STEP = 512


def _fused_body(x_ref, eiT_ref, wlin_ref, att_ref, bias_ref, wcls_ref,
                bcls_ref, out_ref, acc_ref):
    g = pl.program_id(0)
    num_steps = pl.num_programs(0)
    imgs_per_step = ROWS_PER_STEP // IN_CH

    s = jnp.sum(x_ref[...], axis=(1, 2))
    acc_ref[pl.ds(g * imgs_per_step, imgs_per_step), :] = s.reshape(
        imgs_per_step, IN_CH)

    @pl.when(g == num_steps - 1)
    def _epilogue():
        x = acc_ref[...] * (1.0 / SPATIAL)
        h = jnp.dot(x, wlin_ref[...],
                    preferred_element_type=jnp.float32)
        hh = h.reshape(N, HEADS, OUT_CH)
        att_src = att_ref[0:HEADS, :]
        att_dst = att_ref[HEADS:2 * HEADS, :]
        a_src = jnp.sum(hh * att_src[None, :, :], axis=-1)
        a_dst = jnp.sum(hh * att_dst[None, :, :], axis=-1)

        e = a_src[:, None, :] + a_dst[None, :, :]
        e = jnp.where(e >= 0, e, 0.2 * e)

        src = eiT_ref[:, 0:1]
        dst = eiT_ref[:, 1:2]
        num_edges = eiT_ref.shape[0]
        ids = lax.broadcasted_iota(jnp.int32, (num_edges, N), 1)
        oh_s = (src == ids).astype(jnp.float32)
        oh_d = (dst == ids).astype(jnp.float32)
        cmat = lax.dot_general(oh_s, oh_d, (((0,), (0,)), ((), ())),
                               preferred_element_type=jnp.float32)
        present = (cmat > 0.0).astype(jnp.float32)

        e_m = e + (present[:, :, None] - 1.0) * jnp.float32(1e30)
        mx = jnp.max(e_m, axis=0)
        ee = jnp.exp(jnp.minimum(e - mx[None, :, :], 0.0))
        ee = ee * cmat[:, :, None]
        denom = jnp.sum(ee, axis=0) + jnp.float32(1e-16)

        outs = []
        for hd in range(HEADS):
            w = ee[:, :, hd]
            num = lax.dot_general(w, h[:, hd * OUT_CH:(hd + 1) * OUT_CH],
                                  (((0,), (0,)), ((), ())),
                                  preferred_element_type=jnp.float32)
            outs.append(num / denom[:, hd][:, None])
        gat = jnp.concatenate(outs, axis=1) + bias_ref[...]
        act = gat * jax.nn.sigmoid(gat)
        logit = jnp.dot(act, wcls_ref[...],
                        preferred_element_type=jnp.float32) + bcls_ref[...]
        out_ref[...] = logit


def kernel(roi_feats, edge_index, W_lin, att_src, att_dst, bias_gat,
           W_cls, b_cls):
    x3d = roi_feats.reshape(N * IN_CH, 56, 56)
    eiT = edge_index.T
    att = jnp.concatenate([att_src, att_dst], axis=0)
    bias2d = bias_gat.reshape(1, HIDDEN)
    bcls2d = b_cls.reshape(1, NUM_AU)
    grid = (x3d.shape[0] // ROWS_PER_STEP,)

    out = pl.pallas_call(
        _fused_body,
        grid=grid,
        in_specs=[
            pl.BlockSpec((ROWS_PER_STEP, 56, 56), lambda g: (g, 0, 0)),
            pl.BlockSpec(eiT.shape, lambda g: (0, 0)),
            pl.BlockSpec(W_lin.shape, lambda g: (0, 0)),
            pl.BlockSpec(att.shape, lambda g: (0, 0)),
            pl.BlockSpec(bias2d.shape, lambda g: (0, 0)),
            pl.BlockSpec(W_cls.shape, lambda g: (0, 0)),
            pl.BlockSpec(bcls2d.shape, lambda g: (0, 0)),
        ],
        out_specs=pl.BlockSpec((N, NUM_AU), lambda g: (0, 0)),
        out_shape=jax.ShapeDtypeStruct((N, NUM_AU), jnp.float32),
        scratch_shapes=[pltpu.VMEM((N, IN_CH), jnp.float32)],
        compiler_params=pltpu.CompilerParams(
            dimension_semantics=("arbitrary",)),
    )(x3d, eiT, W_lin, att, bias2d, W_cls, bcls2d)
    return out
